# baseline (speedup 1.0000x reference)
import functools

import numpy as np
import jax
import jax.numpy as jnp
from jax import lax
from jax.experimental import pallas as pl
from jax.experimental.pallas import tpu as pltpu
from jax.experimental.pallas import tpu_sc as plsc

D = 16777216
B_TOTAL = 131072
NH = 150
PRIME = 2147483647
_SEED = 42

NC = 2
NS = 16
LANES = 16
NW = NC * NS
KPW = B_TOTAL // NW
CK = 128
NCHUNK = KPW // CK
NVREG = CK // LANES

MASK24 = (1 << 24) - 1

NWORDS = D // 32
EPT = D // NS
FB = 2048
PACK_ITERS = EPT // FB
WPT = EPT // 32
_POW2 = [int(np.int32(np.uint32(1 << r))) for r in range(32)]


def _hash_consts():
    rng = np.random.RandomState(_SEED)
    a = rng.randint(1, PRIME, size=(NH,)).astype(np.int64)
    b = rng.randint(0, PRIME, size=(NH,)).astype(np.int64)
    ah = np.broadcast_to((a >> 16).astype(np.int32)[:, None],
                         (NH, LANES)).reshape(-1).copy()
    al = np.broadcast_to((a & 0xFFFF).astype(np.int32)[:, None],
                         (NH, LANES)).reshape(-1).copy()
    bb = np.broadcast_to(b.astype(np.int32)[:, None],
                         (NH, LANES)).reshape(-1).copy()
    return ah, al, bb


_AH, _AL, _BB = _hash_consts()


def _i32(v):
    return jnp.int32(v)


def _redc(x, e):
    s = 31 - e
    hi = lax.shift_right_logical(x, _i32(s))
    lo = lax.shift_left(jnp.bitwise_and(x, _i32((1 << s) - 1)), _i32(e))
    return hi + lo


def _addmod(x, y):
    s = x + y
    hi = lax.shift_right_logical(s, _i32(31))
    lo = jnp.bitwise_and(s, _i32(0x7FFFFFFF))
    return hi + lo


def _bloom_body(keys_hbm, ah_hbm, al_hbm, bb_hbm, mem_hbm, out_hbm,
                keys_v, ah_v, al_v, bb_v, idx_a, idx_b, rb_a, rb_b,
                vals_v, out_v, fbuf, wbuf, bitmap_sh, sem, psem):
    cid = lax.axis_index("c")
    sid = lax.axis_index("s")
    wid = sid * _i32(NC) + cid
    base = wid * _i32(KPW)
    is_bitmap = sid >= _i32(NS // 2)

    ebase = sid * _i32(EPT)
    wbase = sid * _i32(WPT)

    pltpu.async_copy(mem_hbm.at[pl.ds(ebase, FB)], fbuf.at[_i32(0)], psem)

    def pack_body(it, _):
        nxt = it + _i32(1)

        @pl.when(nxt < _i32(PACK_ITERS))
        def _():
            pltpu.async_copy(mem_hbm.at[pl.ds(ebase + nxt * _i32(FB), FB)],
                             fbuf.at[jnp.bitwise_and(nxt, _i32(1))], psem)

        pltpu.make_async_copy(mem_hbm.at[pl.ds(_i32(0), FB)],
                              fbuf.at[_i32(0)], psem).wait()
        par = jnp.bitwise_and(it, _i32(1))
        for blk in range(FB // 512):
            acc = jnp.zeros((LANES,), jnp.int32)
            for r in range(32):
                v = fbuf[par, pl.ds(blk * 512 + r * LANES, LANES)]
                nz = jnp.bitwise_and(v, _i32(0x7FFFFFFF)) != _i32(0)
                acc = jnp.bitwise_or(
                    acc, jnp.where(nz, _i32(_POW2[r]), _i32(0)))
            wbuf[pl.ds(blk * LANES, LANES)] = acc
        pltpu.sync_copy(wbuf, bitmap_sh.at[pl.ds(wbase + it * _i32(FB // 32),
                                                 FB // 32)])
        return 0

    lax.fori_loop(_i32(0), _i32(PACK_ITERS), pack_body, 0)
    plsc.subcore_barrier()

    pltpu.sync_copy(keys_hbm.at[pl.ds(base, KPW)], keys_v)
    pltpu.sync_copy(ah_hbm, ah_v)
    pltpu.sync_copy(al_hbm, al_v)
    pltpu.sync_copy(bb_hbm, bb_v)

    def compute_idx(c, idx_v, rb_v):
        cbase = c * _i32(CK)
        kvs = [keys_v[pl.ds(cbase + _i32(v * LANES), LANES)]
               for v in range(NVREG)]
        khs = [lax.shift_right_logical(kv, _i32(12)) for kv in kvs]
        kls = [jnp.bitwise_and(kv, _i32(0xFFF)) for kv in kvs]

        def hash_body(j, _):
            joff16 = j * _i32(LANES)
            ah = ah_v[pl.ds(joff16, LANES)]
            al = al_v[pl.ds(joff16, LANES)]
            bb = bb_v[pl.ds(joff16, LANES)]
            r_prev = None
            for v in range(NVREG):
                t1 = _redc(khs[v] * ah, 28)
                t2 = _redc(khs[v] * al, 12)
                t3 = _redc(kls[v] * ah, 16)
                t4 = kls[v] * al
                acc = _addmod(_addmod(t1, t2), _addmod(t3, _addmod(t4, bb)))
                acc = jnp.where(acc == _i32(PRIME), _i32(0), acc)
                idx = jnp.bitwise_and(acc, _i32(MASK24))
                w = jnp.bitwise_or(
                    lax.shift_left(lax.shift_right_logical(idx, _i32(9)),
                                   _i32(4)),
                    jnp.bitwise_and(idx, _i32(0xF)))
                r = jnp.bitwise_and(lax.shift_right_logical(idx, _i32(4)),
                                    _i32(0x1F))
                idx_v[pl.ds(j * _i32(CK) + _i32(v * LANES), LANES)] = \
                    jnp.where(is_bitmap, w, idx)
                if v % 2 == 0:
                    r_prev = r
                else:
                    packed = jnp.bitwise_or(r_prev, lax.shift_left(r, _i32(16)))
                    rb_v[pl.ds(j * _i32(CK // 2) + _i32((v // 2) * LANES),
                               LANES)] = \
                        jnp.where(is_bitmap, packed, _i32(23 | (23 << 16)))
            return 0

        lax.fori_loop(_i32(0), _i32(NH), hash_body, 0)

    GL = NH * CK // 1

    def fire(idx_v):
        @pl.when(is_bitmap)
        def _():
            for g in range(NH * CK // GL):
                pltpu.async_copy(bitmap_sh.at[idx_v.at[pl.ds(g * GL, GL)]],
                                 vals_v.at[pl.ds(g * GL, GL)], sem)

        @pl.when(jnp.logical_not(is_bitmap))
        def _():
            for g in range(NH * CK // GL):
                pltpu.async_copy(mem_hbm.at[idx_v.at[pl.ds(g * GL, GL)]],
                                 vals_v.at[pl.ds(g * GL, GL)], sem)

    def drain():
        pltpu.make_async_copy(mem_hbm.at[pl.ds(_i32(0), NH * CK)],
                              vals_v, sem).wait()

    def reduce(c, rb_v):
        cbase = c * _i32(CK)

        def red_body(j, accs):
            joff = j * _i32(CK)
            out = []
            for p in range(NVREG // 2):
                w0 = vals_v[pl.ds(joff + _i32(2 * p * LANES), LANES)]
                w1 = vals_v[pl.ds(joff + _i32((2 * p + 1) * LANES), LANES)]
                rr = rb_v[pl.ds(j * _i32(CK // 2) + _i32(p * LANES), LANES)]
                r0 = jnp.bitwise_and(rr, _i32(0x1F))
                r1 = lax.shift_right_logical(rr, _i32(16))
                b0 = jnp.bitwise_and(lax.shift_right_logical(w0, r0), _i32(1))
                b1 = jnp.bitwise_and(lax.shift_right_logical(w1, r1), _i32(1))
                out.append(accs[2 * p] + b0)
                out.append(accs[2 * p + 1] + b1)
            return tuple(out)

        zeros = tuple(jnp.zeros((LANES,), jnp.int32) for _ in range(NVREG))
        accs = lax.fori_loop(_i32(0), _i32(NH), red_body, zeros)
        for v in range(NVREG):
            out_v[pl.ds(cbase + _i32(v * LANES), LANES)] = \
                accs[v].astype(jnp.float32) / float(NH)

    compute_idx(_i32(0), idx_a, rb_a)
    fire(idx_a)

    def pair_body(i, _):
        ce = _i32(2) * i
        co = ce + _i32(1)
        cn = ce + _i32(2)
        compute_idx(co, idx_b, rb_b)
        drain()
        reduce(ce, rb_a)
        fire(idx_b)

        @pl.when(cn < _i32(NCHUNK))
        def _():
            compute_idx(cn, idx_a, rb_a)

        drain()
        reduce(co, rb_b)

        @pl.when(cn < _i32(NCHUNK))
        def _():
            fire(idx_a)

        return 0

    lax.fori_loop(_i32(0), _i32(NCHUNK // 2), pair_body, 0)
    pltpu.sync_copy(out_v, out_hbm.at[pl.ds(base, KPW)])


@functools.partial(jax.jit, static_argnums=())
def _bloom_read(keys32, ah, al, bb, mem_i32):
    mesh = plsc.VectorSubcoreMesh(core_axis_name="c", subcore_axis_name="s")
    return pl.kernel(
        _bloom_body,
        out_type=jax.ShapeDtypeStruct((B_TOTAL,), jnp.float32),
        mesh=mesh,
        scratch_types=[
            pltpu.VMEM((KPW,), jnp.int32),
            pltpu.VMEM((NH * LANES,), jnp.int32),
            pltpu.VMEM((NH * LANES,), jnp.int32),
            pltpu.VMEM((NH * LANES,), jnp.int32),
            pltpu.VMEM((NH * CK,), jnp.int32),
            pltpu.VMEM((NH * CK,), jnp.int32),
            pltpu.VMEM((NH * CK // 2,), jnp.int32),
            pltpu.VMEM((NH * CK // 2,), jnp.int32),
            pltpu.VMEM((NH * CK,), jnp.int32),
            pltpu.VMEM((KPW,), jnp.float32),
            pltpu.VMEM((2, FB), jnp.int32),
            pltpu.VMEM((FB // 32,), jnp.int32),
            pltpu.VMEM_SHARED((NWORDS,), jnp.int32),
            pltpu.SemaphoreType.DMA,
            pltpu.SemaphoreType.DMA,
        ],
    )(keys32, ah, al, bb, mem_i32)


def kernel(keys, memory):
    keys32 = keys.astype(jnp.int32)
    mem_i32 = lax.bitcast_convert_type(memory, jnp.int32)
    return _bloom_read(keys32, jnp.asarray(_AH), jnp.asarray(_AL),
                       jnp.asarray(_BB), mem_i32)


# speedup vs baseline: 1.7214x; 1.7214x over previous
import functools

import numpy as np
import jax
import jax.numpy as jnp
from jax import lax
from jax.experimental import pallas as pl
from jax.experimental.pallas import tpu as pltpu
from jax.experimental.pallas import tpu_sc as plsc

D = 16777216
B_TOTAL = 131072
NH = 150
PRIME = 2147483647
_SEED = 42

NC = 2
NS = 16
LANES = 16
NW = NC * NS
KPW = B_TOTAL // NW
CK = 128
NCHUNK = KPW // CK
NVREG = CK // LANES

MASK24 = (1 << 24) - 1

NWORDS = D // 32
EPT = D // NS
FB = 2048
PACK_ITERS = EPT // FB
WPT = EPT // 32
_POW2 = [int(np.int32(np.uint32(1 << r))) for r in range(32)]


def _hash_consts():
    rng = np.random.RandomState(_SEED)
    a = rng.randint(1, PRIME, size=(NH,)).astype(np.int64)
    b = rng.randint(0, PRIME, size=(NH,)).astype(np.int64)
    ah = np.broadcast_to((a >> 16).astype(np.int32)[:, None],
                         (NH, LANES)).reshape(-1).copy()
    al = np.broadcast_to((a & 0xFFFF).astype(np.int32)[:, None],
                         (NH, LANES)).reshape(-1).copy()
    bb = np.broadcast_to(b.astype(np.int32)[:, None],
                         (NH, LANES)).reshape(-1).copy()
    return ah, al, bb


_AH, _AL, _BB = _hash_consts()


def _i32(v):
    return jnp.int32(v)


def _redc(x, e):
    s = 31 - e
    hi = lax.shift_right_logical(x, _i32(s))
    lo = lax.shift_left(jnp.bitwise_and(x, _i32((1 << s) - 1)), _i32(e))
    return hi + lo


def _addmod(x, y):
    s = x + y
    hi = lax.shift_right_logical(s, _i32(31))
    lo = jnp.bitwise_and(s, _i32(0x7FFFFFFF))
    return hi + lo


def _bloom_body(keys_hbm, ah_hbm, al_hbm, bb_hbm, mem_hbm, out_hbm,
                keys_v, ah_v, al_v, bb_v, idx_a, idx_b, rb_a, rb_b,
                vals_v, out_v, fbuf, wbuf, bitmap_sh, sem, psem):
    cid = lax.axis_index("c")
    sid = lax.axis_index("s")
    wid = sid * _i32(NC) + cid
    base = wid * _i32(KPW)

    ebase = sid * _i32(EPT)
    wbase = sid * _i32(WPT)

    pltpu.async_copy(mem_hbm.at[pl.ds(ebase, FB)], fbuf.at[_i32(0)], psem)

    def pack_body(it, _):
        nxt = it + _i32(1)

        @pl.when(nxt < _i32(PACK_ITERS))
        def _():
            pltpu.async_copy(mem_hbm.at[pl.ds(ebase + nxt * _i32(FB), FB)],
                             fbuf.at[jnp.bitwise_and(nxt, _i32(1))], psem)

        pltpu.make_async_copy(mem_hbm.at[pl.ds(_i32(0), FB)],
                              fbuf.at[_i32(0)], psem).wait()
        par = jnp.bitwise_and(it, _i32(1))
        for blk in range(FB // 512):
            acc = jnp.zeros((LANES,), jnp.int32)
            for r in range(32):
                v = fbuf[par, pl.ds(blk * 512 + r * LANES, LANES)]
                acc = jnp.bitwise_or(
                    acc, jnp.where(v != 0.0, _i32(_POW2[r]), _i32(0)))
            wbuf[pl.ds(blk * LANES, LANES)] = acc
        pltpu.sync_copy(wbuf, bitmap_sh.at[pl.ds(wbase + it * _i32(FB // 32),
                                                 FB // 32)])
        return 0

    lax.fori_loop(_i32(0), _i32(PACK_ITERS), pack_body, 0)
    plsc.subcore_barrier()

    pltpu.sync_copy(keys_hbm.at[pl.ds(base, KPW)], keys_v)
    pltpu.sync_copy(ah_hbm, ah_v)
    pltpu.sync_copy(al_hbm, al_v)
    pltpu.sync_copy(bb_hbm, bb_v)

    def compute_idx(c, idx_v, rb_v):
        cbase = c * _i32(CK)
        kvs = [keys_v[pl.ds(cbase + _i32(v * LANES), LANES)]
               for v in range(NVREG)]
        khs = [lax.shift_right_logical(kv, _i32(12)) for kv in kvs]
        kls = [jnp.bitwise_and(kv, _i32(0xFFF)) for kv in kvs]

        def hash_body(j, _):
            joff16 = j * _i32(LANES)
            ah = ah_v[pl.ds(joff16, LANES)]
            al = al_v[pl.ds(joff16, LANES)]
            bb = bb_v[pl.ds(joff16, LANES)]
            r_prev = None
            for v in range(NVREG):
                t1 = _redc(khs[v] * ah, 28)
                t2 = _redc(khs[v] * al, 12)
                t3 = _redc(kls[v] * ah, 16)
                t4 = kls[v] * al
                acc = _addmod(_addmod(t1, t2), _addmod(t3, _addmod(t4, bb)))
                acc = jnp.where(acc == _i32(PRIME), _i32(0), acc)
                idx = jnp.bitwise_and(acc, _i32(MASK24))
                w = jnp.bitwise_or(
                    lax.shift_left(lax.shift_right_logical(idx, _i32(9)),
                                   _i32(4)),
                    jnp.bitwise_and(idx, _i32(0xF)))
                r = jnp.bitwise_and(lax.shift_right_logical(idx, _i32(4)),
                                    _i32(0x1F))
                idx_v[pl.ds(j * _i32(CK) + _i32(v * LANES), LANES)] = w
                if v % 2 == 0:
                    r_prev = r
                else:
                    rb_v[pl.ds(j * _i32(CK // 2) + _i32((v // 2) * LANES),
                               LANES)] = jnp.bitwise_or(
                        r_prev, lax.shift_left(r, _i32(16)))
            return 0

        lax.fori_loop(_i32(0), _i32(NH), hash_body, 0)

    GL = NH * CK // 1

    def fire(idx_v):
        for g in range(NH * CK // GL):
            pltpu.async_copy(bitmap_sh.at[idx_v.at[pl.ds(g * GL, GL)]],
                             vals_v.at[pl.ds(g * GL, GL)], sem)

    def drain():
        pltpu.make_async_copy(keys_hbm.at[pl.ds(_i32(0), NH * CK)],
                              vals_v, sem).wait()

    def reduce(c, rb_v):
        cbase = c * _i32(CK)

        def red_body(j, accs):
            joff = j * _i32(CK)
            out = []
            for p in range(NVREG // 2):
                w0 = vals_v[pl.ds(joff + _i32(2 * p * LANES), LANES)]
                w1 = vals_v[pl.ds(joff + _i32((2 * p + 1) * LANES), LANES)]
                rr = rb_v[pl.ds(j * _i32(CK // 2) + _i32(p * LANES), LANES)]
                r0 = jnp.bitwise_and(rr, _i32(0x1F))
                r1 = lax.shift_right_logical(rr, _i32(16))
                b0 = jnp.bitwise_and(lax.shift_right_logical(w0, r0), _i32(1))
                b1 = jnp.bitwise_and(lax.shift_right_logical(w1, r1), _i32(1))
                out.append(accs[2 * p] + b0)
                out.append(accs[2 * p + 1] + b1)
            return tuple(out)

        zeros = tuple(jnp.zeros((LANES,), jnp.int32) for _ in range(NVREG))
        accs = lax.fori_loop(_i32(0), _i32(NH), red_body, zeros)
        for v in range(NVREG):
            out_v[pl.ds(cbase + _i32(v * LANES), LANES)] = \
                accs[v].astype(jnp.float32) / float(NH)

    compute_idx(_i32(0), idx_a, rb_a)
    fire(idx_a)

    def pair_body(i, _):
        ce = _i32(2) * i
        co = ce + _i32(1)
        cn = ce + _i32(2)
        compute_idx(co, idx_b, rb_b)
        drain()
        reduce(ce, rb_a)
        fire(idx_b)

        @pl.when(cn < _i32(NCHUNK))
        def _():
            compute_idx(cn, idx_a, rb_a)

        drain()
        reduce(co, rb_b)

        @pl.when(cn < _i32(NCHUNK))
        def _():
            fire(idx_a)

        return 0

    lax.fori_loop(_i32(0), _i32(NCHUNK // 2), pair_body, 0)
    pltpu.sync_copy(out_v, out_hbm.at[pl.ds(base, KPW)])


@functools.partial(jax.jit, static_argnums=())
def _bloom_read(keys32, ah, al, bb, memory):
    mesh = plsc.VectorSubcoreMesh(core_axis_name="c", subcore_axis_name="s")
    return pl.kernel(
        _bloom_body,
        out_type=jax.ShapeDtypeStruct((B_TOTAL,), jnp.float32),
        mesh=mesh,
        scratch_types=[
            pltpu.VMEM((KPW,), jnp.int32),
            pltpu.VMEM((NH * LANES,), jnp.int32),
            pltpu.VMEM((NH * LANES,), jnp.int32),
            pltpu.VMEM((NH * LANES,), jnp.int32),
            pltpu.VMEM((NH * CK,), jnp.int32),
            pltpu.VMEM((NH * CK,), jnp.int32),
            pltpu.VMEM((NH * CK // 2,), jnp.int32),
            pltpu.VMEM((NH * CK // 2,), jnp.int32),
            pltpu.VMEM((NH * CK,), jnp.int32),
            pltpu.VMEM((KPW,), jnp.float32),
            pltpu.VMEM((2, FB), jnp.float32),
            pltpu.VMEM((FB // 32,), jnp.int32),
            pltpu.VMEM_SHARED((NWORDS,), jnp.int32),
            pltpu.SemaphoreType.DMA,
            pltpu.SemaphoreType.DMA,
        ],
    )(keys32, ah, al, bb, memory)


def kernel(keys, memory):
    keys32 = keys.astype(jnp.int32)
    return _bloom_read(keys32, jnp.asarray(_AH), jnp.asarray(_AL),
                       jnp.asarray(_BB), memory)
